# speedup vs baseline: 2.7597x; 2.7597x over previous
import jax
import jax.numpy as jnp
from jax.experimental import pallas as pl


def _copy_body(in_ref, out_ref):
    out_ref[...] = in_ref[...]


def kernel(x, pos_emb_table):
    rows, d = pos_emb_table.shape
    block_rows = 512
    out = pl.pallas_call(
        _copy_body,
        grid=(rows // block_rows,),
        in_specs=[pl.BlockSpec((block_rows, d), lambda i: (i, 0))],
        out_specs=pl.BlockSpec((block_rows, d), lambda i: (i, 0)),
        out_shape=jax.ShapeDtypeStruct((rows, d), pos_emb_table.dtype),
    )(pos_emb_table)
    return out[None]
